# initial kernel scaffold (unmeasured)
import jax
import jax.numpy as jnp
from jax import lax
from jax.experimental import pallas as pl
from jax.experimental.pallas import tpu as pltpu

N_DEV = 4


def kernel(x, w_mat):
    m_glob, k_loc = x.shape
    k, n = w_mat.shape
    m_loc = m_glob // N_DEV

    def body(x_ref, w_ref, out_ref, comm_ref, blk_send, blk_recv,
             amax_ref, amax_send, amax_recv):
        my = lax.axis_index("i")

        barrier = pltpu.get_barrier_semaphore()
        for d in range(1, N_DEV):
            peer = lax.rem(my + d, N_DEV)
            pl.semaphore_signal(barrier, inc=1, device_id=(peer,),
                                device_id_type=pl.DeviceIdType.MESH)
        pl.semaphore_wait(barrier, N_DEV - 1)

        sends = []
        for d in range(1, N_DEV):
            tgt = lax.rem(my + d, N_DEV)
            rdma = pltpu.make_async_remote_copy(
                src_ref=x_ref.at[pl.ds(tgt * m_loc, m_loc), :],
                dst_ref=comm_ref.at[d - 1],
                send_sem=blk_send.at[d - 1],
                recv_sem=blk_recv.at[d - 1],
                device_id=(tgt,),
                device_id_type=pl.DeviceIdType.MESH,
            )
            rdma.start()
            sends.append(rdma)

        out_ref[...] = jnp.dot(
            x_ref[pl.ds(my * m_loc, m_loc), :],
            w_ref[pl.ds(my * k_loc, k_loc), :],
            preferred_element_type=jnp.float32,
        )

        for d in range(1, N_DEV):
            sends[d - 1].wait_recv()
            src = lax.rem(my - d + N_DEV, N_DEV)
            out_ref[...] += jnp.dot(
                comm_ref[d - 1],
                w_ref[pl.ds(src * k_loc, k_loc), :],
                preferred_element_type=jnp.float32,
            )

        local_amax = jnp.max(jnp.abs(out_ref[...]))
        amax_ref[0] = jnp.full((8, 128), local_amax, jnp.float32)
        amax_sends = []
        for d in range(1, N_DEV):
            tgt = lax.rem(my + d, N_DEV)
            rdma = pltpu.make_async_remote_copy(
                src_ref=amax_ref.at[0],
                dst_ref=amax_ref.at[d],
                send_sem=amax_send.at[d - 1],
                recv_sem=amax_recv.at[d - 1],
                device_id=(tgt,),
                device_id_type=pl.DeviceIdType.MESH,
            )
            rdma.start()
            amax_sends.append(rdma)
        for d in range(1, N_DEV):
            amax_sends[d - 1].wait_recv()

        gmax = jnp.max(amax_ref[...])
        scale = gmax / 448.0
        q = jnp.clip(out_ref[...] / scale, -448.0, 448.0)
        out_ref[...] = q.astype(jnp.float8_e4m3fn).astype(jnp.float32) * scale

        for r in sends + amax_sends:
            r.wait_send()

    return pl.pallas_call(
        body,
        out_shape=jax.ShapeDtypeStruct((m_loc, n), jnp.float32),
        in_specs=[
            pl.BlockSpec(memory_space=pltpu.VMEM),
            pl.BlockSpec(memory_space=pltpu.VMEM),
        ],
        out_specs=pl.BlockSpec(memory_space=pltpu.VMEM),
        scratch_shapes=[
            pltpu.VMEM((N_DEV - 1, m_loc, k_loc), jnp.float32),
            pltpu.SemaphoreType.DMA((N_DEV - 1,)),
            pltpu.SemaphoreType.DMA((N_DEV - 1,)),
            pltpu.VMEM((N_DEV, 8, 128), jnp.float32),
            pltpu.SemaphoreType.DMA((N_DEV - 1,)),
            pltpu.SemaphoreType.DMA((N_DEV - 1,)),
        ],
        compiler_params=pltpu.CompilerParams(collective_id=0),
    )(x, w_mat)


# baseline (device time: 124542 ns/iter reference)
import jax
import jax.numpy as jnp
from jax import lax
from jax.experimental import pallas as pl
from jax.experimental.pallas import tpu as pltpu

N_DEV = 4


def kernel(x, w_mat):
    m_glob, k_loc = x.shape
    k, n = w_mat.shape
    m_loc = m_glob // N_DEV

    def body(x_ref, w_ref, out_ref, comm_ref, blk_send, blk_recv,
             w_buf, w_sems, amax_ref, amax_send, amax_recv):
        my = lax.axis_index("i")

        def w_block(step):
            src = lax.rem(my - step + N_DEV, N_DEV)
            copy = pltpu.make_async_copy(
                w_ref.at[pl.ds(src * k_loc, k_loc), :],
                w_buf.at[step % 2],
                w_sems.at[step % 2],
            )
            copy.start()
            return copy

        barrier = pltpu.get_barrier_semaphore()
        for d in range(1, N_DEV):
            peer = lax.rem(my + d, N_DEV)
            pl.semaphore_signal(barrier, inc=1, device_id=(peer,),
                                device_id_type=pl.DeviceIdType.MESH)
        pl.semaphore_wait(barrier, N_DEV - 1)

        sends = []
        for d in range(1, N_DEV):
            tgt = lax.rem(my + d, N_DEV)
            rdma = pltpu.make_async_remote_copy(
                src_ref=x_ref.at[pl.ds(tgt * m_loc, m_loc), :],
                dst_ref=comm_ref.at[d - 1],
                send_sem=blk_send.at[d - 1],
                recv_sem=blk_recv.at[d - 1],
                device_id=(tgt,),
                device_id_type=pl.DeviceIdType.MESH,
            )
            rdma.start()
            sends.append(rdma)

        w_copies = [w_block(0), w_block(1)]
        w_copies[0].wait()
        out_ref[...] = jnp.dot(
            x_ref[pl.ds(my * m_loc, m_loc), :],
            w_buf[0],
            preferred_element_type=jnp.float32,
        )

        for d in range(1, N_DEV):
            if d + 1 < N_DEV:
                w_copies.append(w_block(d + 1))
            sends[d - 1].wait_recv()
            w_copies[d].wait()
            out_ref[...] += jnp.dot(
                comm_ref[d - 1],
                w_buf[d % 2],
                preferred_element_type=jnp.float32,
            )

        local_amax = jnp.max(jnp.abs(out_ref[...]))
        amax_ref[0] = jnp.full((8, 128), local_amax, jnp.float32)
        amax_sends = []
        for d in range(1, N_DEV):
            tgt = lax.rem(my + d, N_DEV)
            rdma = pltpu.make_async_remote_copy(
                src_ref=amax_ref.at[0],
                dst_ref=amax_ref.at[d],
                send_sem=amax_send.at[d - 1],
                recv_sem=amax_recv.at[d - 1],
                device_id=(tgt,),
                device_id_type=pl.DeviceIdType.MESH,
            )
            rdma.start()
            amax_sends.append(rdma)
        for d in range(1, N_DEV):
            amax_sends[d - 1].wait_recv()

        gmax = jnp.max(amax_ref[...])
        scale = gmax / 448.0
        q = jnp.clip(out_ref[...] / scale, -448.0, 448.0)
        out_ref[...] = q.astype(jnp.float8_e4m3fn).astype(jnp.float32) * scale

        for r in sends + amax_sends:
            r.wait_send()

    return pl.pallas_call(
        body,
        out_shape=jax.ShapeDtypeStruct((m_loc, n), jnp.float32),
        in_specs=[
            pl.BlockSpec(memory_space=pltpu.VMEM),
            pl.BlockSpec(memory_space=pl.ANY),
        ],
        out_specs=pl.BlockSpec(memory_space=pltpu.VMEM),
        scratch_shapes=[
            pltpu.VMEM((N_DEV - 1, m_loc, k_loc), jnp.float32),
            pltpu.SemaphoreType.DMA((N_DEV - 1,)),
            pltpu.SemaphoreType.DMA((N_DEV - 1,)),
            pltpu.VMEM((2, k_loc, n), jnp.float32),
            pltpu.SemaphoreType.DMA((2,)),
            pltpu.VMEM((N_DEV, 8, 128), jnp.float32),
            pltpu.SemaphoreType.DMA((N_DEV - 1,)),
            pltpu.SemaphoreType.DMA((N_DEV - 1,)),
        ],
        compiler_params=pltpu.CompilerParams(
            collective_id=0,
            vmem_limit_bytes=63 * 1024 * 1024,
        ),
    )(x, w_mat)
